# baseline (speedup 1.0000x reference)
import functools

import jax
import jax.numpy as jnp
from jax import lax
from jax.experimental import pallas as pl
from jax.experimental.pallas import tpu as pltpu
from jax.experimental.pallas import tpu_sc as plsc

N_NODES_ = 10000
HID = 128
N_EDGES_ = 320000

NC = 2
NS = 16
NW = NC * NS
LANES = 16

CHUNK = 400
UNROLL = 4


def _sc_edge_dots(z, src, dst):
  B = src.shape[0]
  per_w = B // NW
  n_chunks = per_w // CHUNK
  mesh = plsc.VectorSubcoreMesh(core_axis_name="c", subcore_axis_name="s",
                                num_cores=NC, num_subcores=NS)

  @functools.partial(
      pl.kernel,
      out_type=jax.ShapeDtypeStruct((B,), jnp.float32),
      mesh=mesh,
      compiler_params=pltpu.CompilerParams(needs_layout_passes=False),
      scratch_types=[
          pltpu.VMEM((CHUNK,), jnp.int32),
          pltpu.VMEM((CHUNK,), jnp.int32),
          pltpu.VMEM((CHUNK, HID), jnp.float32),
          pltpu.VMEM((CHUNK, HID), jnp.float32),
          pltpu.VMEM((CHUNK,), jnp.float32),
          pltpu.SemaphoreType.DMA,
          pltpu.SemaphoreType.DMA,
      ],
  )
  def sc_kernel(z_hbm, src_hbm, dst_hbm, out_hbm,
                si_v, di_v, a_v, b_v, d_v, sem_a, sem_b):
    wid = lax.axis_index("s") * NC + lax.axis_index("c")
    base = wid * per_w

    def chunk_body(c, carry):
      off = base + c * CHUNK
      pltpu.sync_copy(src_hbm.at[pl.ds(off, CHUNK)], si_v)
      pltpu.sync_copy(dst_hbm.at[pl.ds(off, CHUNK)], di_v)
      cp_a = pltpu.async_copy(z_hbm.at[si_v], a_v, sem_a)
      cp_b = pltpu.async_copy(z_hbm.at[di_v], b_v, sem_b)
      cp_a.wait()
      cp_b.wait()

      def edge_body(g, carry2):
        e_vec = g * LANES + lax.iota(jnp.int32, LANES)
        NACC = 8
        accs = [jnp.zeros((LANES,), jnp.float32) for _ in range(NACC)]
        for kk in range(HID):
          k_vec = jnp.full((LANES,), kk, jnp.int32)
          va = plsc.load_gather(a_v, [e_vec, k_vec])
          vb = plsc.load_gather(b_v, [e_vec, k_vec])
          accs[kk % NACC] = accs[kk % NACC] + va * vb
        while len(accs) > 1:
          accs = [accs[i] + accs[i + 1] for i in range(0, len(accs), 2)]
        d_v[pl.ds(g * LANES, LANES)] = accs[0]
        return carry2

      lax.fori_loop(0, CHUNK // LANES, edge_body, 0)
      pltpu.sync_copy(d_v, out_hbm.at[pl.ds(off, CHUNK)])
      return carry

    lax.fori_loop(0, n_chunks, chunk_body, 0)

  return sc_kernel(z, src, dst)


def _tc_loss_kernel(v_ref, out_ref):
  EPS = 1e-15
  x = v_ref[:]
  half = x.shape[0] // 2
  s = 1.0 / (1.0 + jnp.exp(-x))
  pos_sum = jnp.sum(jnp.log(s[:half] + EPS))
  neg_sum = jnp.sum(jnp.log(1.0 - s[half:] + EPS))
  out_ref[0, 0] = pos_sum
  out_ref[0, 1] = neg_sum


def kernel(z, edge_index, neg_edge_index):
  n = z.shape[0]
  E = edge_index.shape[1]
  EPSW = float(n * n - 2) / 2.0
  norm = n * n / float((n * n - 2) * 2)

  src = jnp.concatenate([edge_index[0], neg_edge_index[0]]).astype(jnp.int32)
  dst = jnp.concatenate([edge_index[1], neg_edge_index[1]]).astype(jnp.int32)

  dots = _sc_edge_dots(z, src, dst)

  v2d = dots.reshape(2 * E // HID, HID)
  sums = pl.pallas_call(
      _tc_loss_kernel,
      out_shape=jax.ShapeDtypeStruct((1, 2), jnp.float32),
      in_specs=[pl.BlockSpec(memory_space=pltpu.VMEM)],
      out_specs=pl.BlockSpec(memory_space=pltpu.SMEM),
  )(v2d)

  pos_loss = -sums[0, 0] / E
  neg_loss = -sums[0, 1] / E
  return norm * (pos_loss * EPSW + neg_loss)


# speedup vs baseline: 1.1378x; 1.1378x over previous
import functools

import jax
import jax.numpy as jnp
from jax import lax
from jax.experimental import pallas as pl
from jax.experimental.pallas import tpu as pltpu
from jax.experimental.pallas import tpu_sc as plsc

N_NODES_ = 10000
HID = 128
N_EDGES_ = 320000

NC = 2
NS = 16
NW = NC * NS
LANES = 16

CHUNK = 400
UNROLL = 4


def _sc_edge_dots(z, src, dst):
  B = src.shape[0]
  per_w = B // NW
  n_chunks = per_w // CHUNK
  mesh = plsc.VectorSubcoreMesh(core_axis_name="c", subcore_axis_name="s",
                                num_cores=NC, num_subcores=NS)

  @functools.partial(
      pl.kernel,
      out_type=jax.ShapeDtypeStruct((B,), jnp.float32),
      mesh=mesh,
      compiler_params=pltpu.CompilerParams(needs_layout_passes=False),
      scratch_types=[
          pltpu.VMEM((CHUNK,), jnp.int32),
          pltpu.VMEM((CHUNK,), jnp.int32),
          pltpu.VMEM((CHUNK, HID), jnp.float32),
          pltpu.VMEM((CHUNK, HID), jnp.float32),
          pltpu.VMEM((CHUNK,), jnp.float32),
          pltpu.SemaphoreType.DMA,
          pltpu.SemaphoreType.DMA,
      ],
  )
  def sc_kernel(z_hbm, src_hbm, dst_hbm, out_hbm,
                si_v, di_v, a_v, b_v, d_v, sem_a, sem_b):
    wid = lax.axis_index("s") * NC + lax.axis_index("c")
    base = wid * per_w

    def chunk_body(c, carry):
      off = base + c * CHUNK
      pltpu.sync_copy(src_hbm.at[pl.ds(off, CHUNK)], si_v)
      pltpu.sync_copy(dst_hbm.at[pl.ds(off, CHUNK)], di_v)
      cp_a = pltpu.async_copy(z_hbm.at[si_v], a_v, sem_a)
      cp_b = pltpu.async_copy(z_hbm.at[di_v], b_v, sem_b)
      cp_a.wait()
      cp_b.wait()

      def edge_body(g, carry2):
        e_vec = g * LANES + lax.iota(jnp.int32, LANES)
        acc = jnp.zeros((LANES,), jnp.float32)
        for kk in range(HID):
          k_vec = jnp.full((LANES,), kk, jnp.int32)
          va = plsc.load_gather(a_v, [e_vec, k_vec])
          vb = plsc.load_gather(b_v, [e_vec, k_vec])
          acc = acc + va * vb
        d_v[pl.ds(g * LANES, LANES)] = acc
        return carry2

      lax.fori_loop(0, CHUNK // LANES, edge_body, 0)
      pltpu.sync_copy(d_v, out_hbm.at[pl.ds(off, CHUNK)])
      return carry

    lax.fori_loop(0, n_chunks, chunk_body, 0)

  return sc_kernel(z, src, dst)


def _tc_loss_kernel(v_ref, out_ref):
  EPS = 1e-15
  x = v_ref[:]
  half = x.shape[0] // 2
  s = 1.0 / (1.0 + jnp.exp(-x))
  pos_sum = jnp.sum(jnp.log(s[:half] + EPS))
  neg_sum = jnp.sum(jnp.log(1.0 - s[half:] + EPS))
  out_ref[0, 0] = pos_sum
  out_ref[0, 1] = neg_sum


def kernel(z, edge_index, neg_edge_index):
  n = z.shape[0]
  E = edge_index.shape[1]
  EPSW = float(n * n - 2) / 2.0
  norm = n * n / float((n * n - 2) * 2)

  src = jnp.concatenate([edge_index[0], neg_edge_index[0]]).astype(jnp.int32)
  dst = jnp.concatenate([edge_index[1], neg_edge_index[1]]).astype(jnp.int32)

  dots = _sc_edge_dots(z, src, dst)

  v2d = dots.reshape(2 * E // HID, HID)
  sums = pl.pallas_call(
      _tc_loss_kernel,
      out_shape=jax.ShapeDtypeStruct((1, 2), jnp.float32),
      in_specs=[pl.BlockSpec(memory_space=pltpu.VMEM)],
      out_specs=pl.BlockSpec(memory_space=pltpu.SMEM),
  )(v2d)

  pos_loss = -sums[0, 0] / E
  neg_loss = -sums[0, 1] / E
  return norm * (pos_loss * EPSW + neg_loss)
